# Initial kernel scaffold; baseline (speedup 1.0000x reference)
import functools
import math

import jax
import jax.numpy as jnp
from jax import lax
from jax.experimental import pallas as pl
from jax.experimental.pallas import tpu as pltpu
from jax.experimental.pallas import tpu_sc as plsc

EMB = 32
SCALE = math.sqrt(float(EMB))
LANES = 16

NUM_WORKERS = 32
CHUNK = 1280


def _emb_body(tok_hbm, table_hbm, out_hbm, idx_v, rows_v, sem, *, bpw, nchunk):
    wid = lax.axis_index("s") * 2 + lax.axis_index("c")
    base = wid * bpw

    def chunk_body(c, _):
        off = base + c * CHUNK
        pltpu.sync_copy(tok_hbm.at[pl.ds(off, CHUNK)], idx_v)
        pltpu.async_copy(table_hbm.at[idx_v], rows_v, sem).wait()

        def scale_body(j, _):
            rows_v[j, pl.ds(0, LANES)] = rows_v[j, pl.ds(0, LANES)] * SCALE
            rows_v[j, pl.ds(LANES, LANES)] = (
                rows_v[j, pl.ds(LANES, LANES)] * SCALE)
            return 0

        lax.fori_loop(0, CHUNK, scale_body, 0)
        pltpu.sync_copy(rows_v, out_hbm.at[pl.ds(off, CHUNK)])
        return 0

    lax.fori_loop(0, nchunk, chunk_body, 0)


@jax.jit
def kernel(tokens, embedding_weight):
    b, s = tokens.shape
    total = b * s
    bpw = total // NUM_WORKERS
    nchunk = bpw // CHUNK
    assert bpw % CHUNK == 0

    tok_flat = tokens.reshape(total).astype(jnp.int32)
    mesh = plsc.VectorSubcoreMesh(core_axis_name="c", subcore_axis_name="s")
    run = functools.partial(
        pl.kernel,
        mesh=mesh,
        out_type=jax.ShapeDtypeStruct((total, EMB), jnp.float32),
        scratch_types=[
            pltpu.VMEM((CHUNK,), jnp.int32),
            pltpu.VMEM((CHUNK, EMB), jnp.float32),
            pltpu.SemaphoreType.DMA,
        ],
    )(functools.partial(_emb_body, bpw=bpw, nchunk=nchunk))
    out = run(tok_flat, embedding_weight)
    return out.reshape(b, s, EMB)


# speedup vs baseline: 1.3004x; 1.3004x over previous
import functools
import math

import jax
import jax.numpy as jnp
from jax import lax
from jax.experimental import pallas as pl
from jax.experimental.pallas import tpu as pltpu
from jax.experimental.pallas import tpu_sc as plsc

EMB = 32
SCALE = math.sqrt(float(EMB))
LANES = 16

NUM_WORKERS = 32
CHUNK = 1280


def _emb_body(tok_hbm, table_hbm, out_hbm, idx_v, rows_v, sem, *, bpw, nchunk):
    wid = lax.axis_index("s") * 2 + lax.axis_index("c")
    base = wid * bpw

    def chunk_body(c, _):
        off = base + c * CHUNK
        pltpu.sync_copy(tok_hbm.at[pl.ds(off, CHUNK)], idx_v)
        pltpu.async_copy(table_hbm.at[idx_v], rows_v, sem).wait()

        def scale_body(j, _):
            rows_v[j, pl.ds(0, LANES)] = rows_v[j, pl.ds(0, LANES)] * SCALE
            rows_v[j, pl.ds(LANES, LANES)] = (
                rows_v[j, pl.ds(LANES, LANES)] * SCALE)
            return 0

        lax.fori_loop(0, CHUNK, scale_body, 0)
        pltpu.sync_copy(rows_v, out_hbm.at[pl.ds(off, CHUNK)])
        return 0

    lax.fori_loop(0, nchunk, chunk_body, 0)


@jax.jit
def kernel(tokens, embedding_weight):
    b, s = tokens.shape
    total = b * s
    bpw = total // NUM_WORKERS
    nchunk = bpw // CHUNK
    assert bpw % CHUNK == 0

    tok_flat = tokens.reshape(total).astype(jnp.int32)
    mesh = plsc.VectorSubcoreMesh(core_axis_name="c", subcore_axis_name="s")
    run = functools.partial(
        pl.kernel,
        mesh=mesh,
        out_type=jax.ShapeDtypeStruct((total, EMB), jnp.float32),
        scratch_types=[
            pltpu.VMEM((CHUNK,), jnp.int32),
            pltpu.VMEM((CHUNK, EMB), jnp.float32),
            pltpu.SemaphoreType.DMA,
        ],
        compiler_params=pltpu.CompilerParams(use_tc_tiling_on_sc=False),
    )(functools.partial(_emb_body, bpw=bpw, nchunk=nchunk))
    out = run(tok_flat, embedding_weight)
    return out.reshape(b, s, EMB)
